# Initial kernel scaffold; baseline (speedup 1.0000x reference)
import jax
import jax.numpy as jnp
from jax.experimental import pallas as pl


def kernel(x, emb):
    raise NotImplementedError("write your pallas kernel here")


# speedup vs baseline: 1.6865x; 1.6865x over previous
import functools

import jax
import jax.numpy as jnp
from jax import lax
from jax.experimental import pallas as pl
from jax.experimental.pallas import tpu as pltpu
from jax.experimental.pallas import tpu_sc as plsc

ESIZE = 64
CHUNK = 128

_info = plsc.get_sparse_core_info()
NC, NS = _info.num_cores, _info.num_subcores
NW = NC * NS


@functools.partial(jax.jit, static_argnames=("nch",))
def _gather_rows(idx, emb, nch):
    n_rows = NW * nch * CHUNK
    mesh = plsc.VectorSubcoreMesh(core_axis_name="c", subcore_axis_name="s")

    @functools.partial(
        pl.kernel,
        out_type=jax.ShapeDtypeStruct((n_rows, ESIZE), jnp.float32),
        mesh=mesh,
        scratch_types=[
            pltpu.VMEM((nch, CHUNK), jnp.int32),
            pltpu.VMEM((CHUNK, ESIZE), jnp.float32),
            pltpu.SemaphoreType.DMA,
        ],
        compiler_params=pltpu.CompilerParams(use_tc_tiling_on_sc=False),
    )
    def k(emb_hbm, idx_hbm, out_hbm, idx_v, rows_v, sem):
        wid = lax.axis_index("s") * NC + lax.axis_index("c")
        base = wid * (nch * CHUNK)
        pltpu.sync_copy(idx_hbm.at[wid], idx_v)

        def step(c, _):
            pltpu.async_copy(emb_hbm.at[idx_v.at[c]], rows_v, sem).wait()
            pltpu.sync_copy(rows_v, out_hbm.at[pl.ds(base + c * CHUNK, CHUNK)])
            return _

        lax.fori_loop(0, nch, step, None)

    return k(emb, idx)


def kernel(x, emb):
    orig_shape = x.shape
    flat = x.reshape(-1).astype(jnp.int32)
    n = flat.shape[0]
    assert n % (NW * CHUNK) == 0, n
    nch = n // (NW * CHUNK)
    idx = flat.reshape(NW, nch, CHUNK)
    out = _gather_rows(idx, emb, nch)
    return out.reshape(*orig_shape, ESIZE)


# speedup vs baseline: 1.8736x; 1.1110x over previous
import functools

import jax
import jax.numpy as jnp
from jax import lax
from jax.experimental import pallas as pl
from jax.experimental.pallas import tpu as pltpu
from jax.experimental.pallas import tpu_sc as plsc

ESIZE = 64
CHUNK = 128
NBUF = 8

_info = plsc.get_sparse_core_info()
NC, NS = _info.num_cores, _info.num_subcores
NW = NC * NS


@functools.partial(jax.jit, static_argnames=("nch",))
def _gather_rows(idx, emb, nch):
    n_rows = NW * nch * CHUNK
    ngrp = nch // NBUF
    assert ngrp * NBUF == nch
    mesh = plsc.VectorSubcoreMesh(core_axis_name="c", subcore_axis_name="s")

    @functools.partial(
        pl.kernel,
        out_type=jax.ShapeDtypeStruct((n_rows, ESIZE), jnp.float32),
        mesh=mesh,
        scratch_types=[
            pltpu.VMEM((nch, CHUNK), jnp.int32),
            pltpu.VMEM((NBUF, CHUNK, ESIZE), jnp.float32),
            pltpu.SemaphoreType.DMA((NBUF,)),
            pltpu.SemaphoreType.DMA((NBUF,)),
        ],
        compiler_params=pltpu.CompilerParams(use_tc_tiling_on_sc=False),
    )
    def k(emb_hbm, idx_hbm, out_hbm, idx_v, rows_v, gsem, ssem):
        wid = lax.axis_index("s") * NC + lax.axis_index("c")
        base = wid * (nch * CHUNK)
        pltpu.sync_copy(idx_hbm.at[wid], idx_v)

        def gather(c, b):
            return pltpu.make_async_copy(
                emb_hbm.at[idx_v.at[c]], rows_v.at[b], gsem.at[b]
            )

        def store(c, b):
            return pltpu.make_async_copy(
                rows_v.at[b], out_hbm.at[pl.ds(base + c * CHUNK, CHUNK)], ssem.at[b]
            )

        for b in range(NBUF):
            gather(b, b).start()

        def group(g, _):
            c0 = g * NBUF
            for b in range(NBUF):
                c = c0 + b
                gather(c, b).wait()
                store(c, b).start()
                store(c, b).wait()
                gather(c + NBUF, b).start()
            return _

        lax.fori_loop(0, ngrp - 1, group, None)

        c0 = (ngrp - 1) * NBUF
        for b in range(NBUF):
            c = c0 + b
            gather(c, b).wait()
            store(c, b).start()
            store(c, b).wait()

    return k(emb, idx)


def kernel(x, emb):
    orig_shape = x.shape
    flat = x.reshape(-1).astype(jnp.int32)
    n = flat.shape[0]
    assert n % (NW * CHUNK * NBUF) == 0, n
    nch = n // (NW * CHUNK)
    idx = flat.reshape(NW, nch, CHUNK)
    out = _gather_rows(idx, emb, nch)
    return out.reshape(*orig_shape, ESIZE)
